# baseline (device time: 521274 ns/iter reference)
import functools

import jax
import jax.numpy as jnp
from jax import lax
from jax.experimental import pallas as pl
from jax.experimental.pallas import tpu as pltpu

N_DEV = 16
NXS = 5
NKS = 3


def kernel(x, Wq, Wo, K_ext, V_ext):
    B_loc, Sq, D = x.shape
    B, Skv, _, Dh = K_ext.shape
    H_loc = Wq.shape[1] // Dh
    R = B_loc * Sq
    Dp = 128
    DP = H_loc * Dp
    scale = 1.0 / (Dh ** 0.5)
    f32 = jnp.float32

    def body(x_ref, wq_ref, wo_ref, k_ref, v_ref, out_ref,
             xslots, rsbuf, wqpad, wopad, obufpad,
             accs, pown,
             x_send_sem, acc_send_sem, x_recv_sems, rs_recv_sems):
        my = lax.axis_index("i")
        left = lax.rem(my + N_DEV - 1, N_DEV)
        right = lax.rem(my + 1, N_DEV)

        wqpad[:, :] = jnp.zeros((D, DP), f32)
        wopad[:, :] = jnp.zeros((DP, D), f32)
        obufpad[:, :] = jnp.zeros((R, DP), f32)
        for h in range(H_loc):
            wqpad[:, h * Dp:h * Dp + Dh] = wq_ref[:, h * Dh:(h + 1) * Dh]
            wopad[h * Dp:h * Dp + Dh, :] = wo_ref[h * Dh:(h + 1) * Dh, :]

        def compute_partial(xc, c):
            qpad = jnp.dot(xc, wqpad[:, :], preferred_element_type=f32)
            for b in range(B_loc):
                for h in range(H_loc):
                    qbh = qpad[b * Sq:(b + 1) * Sq, h * Dp:h * Dp + Dh]
                    kbh = k_ref[(c * B_loc + b) * H_loc + h]
                    vbh = v_ref[(c * B_loc + b) * H_loc + h]
                    sc = lax.dot_general(
                        qbh, kbh, (((1,), (1,)), ((), ())),
                        preferred_element_type=f32) * scale
                    m = jnp.max(sc, axis=1, keepdims=True)
                    p = jnp.exp(sc - m)
                    l = jnp.sum(p, axis=1, keepdims=True)
                    o = jnp.dot(p, vbh, preferred_element_type=f32) / l
                    obufpad[b * Sq:(b + 1) * Sq, h * Dp:h * Dp + Dh] = o
            return jnp.dot(obufpad[:, :], wopad[:, :],
                           preferred_element_type=f32)

        def xsend(t, src):
            rdma = pltpu.make_async_remote_copy(
                src_ref=src,
                dst_ref=xslots.at[lax.rem(t + 1, NXS)],
                send_sem=x_send_sem,
                recv_sem=x_recv_sems.at[t],
                device_id=(right,), device_id_type=pl.DeviceIdType.MESH)
            rdma.start()
            return rdma

        def xwait_recv(t):
            pltpu.make_async_remote_copy(
                src_ref=xslots.at[lax.rem(t, NXS)],
                dst_ref=xslots.at[lax.rem(t, NXS)],
                send_sem=x_send_sem, recv_sem=x_recv_sems.at[t - 1],
                device_id=(left,), device_id_type=pl.DeviceIdType.MESH,
            ).wait_recv()

        def xdrain():
            pltpu.make_async_remote_copy(
                src_ref=xslots.at[0], dst_ref=xslots.at[0],
                send_sem=x_send_sem, recv_sem=x_recv_sems.at[0],
                device_id=(right,), device_id_type=pl.DeviceIdType.MESH,
            ).wait_send()

        def acc_send(t, src):
            rdma = pltpu.make_async_remote_copy(
                src_ref=src, dst_ref=rsbuf.at[t - 1],
                send_sem=acc_send_sem, recv_sem=rs_recv_sems.at[t - 1],
                device_id=(right,), device_id_type=pl.DeviceIdType.MESH)
            rdma.start()

        def acc_wait_recv(t):
            pltpu.make_async_remote_copy(
                src_ref=rsbuf.at[t - 2], dst_ref=rsbuf.at[t - 2],
                send_sem=acc_send_sem, recv_sem=rs_recv_sems.at[t - 2],
                device_id=(left,), device_id_type=pl.DeviceIdType.MESH,
            ).wait_recv()

        def acc_drain():
            pltpu.make_async_remote_copy(
                src_ref=accs, dst_ref=accs,
                send_sem=acc_send_sem, recv_sem=rs_recv_sems.at[0],
                device_id=(right,), device_id_type=pl.DeviceIdType.MESH,
            ).wait_send()

        def chunk_of(t):
            return lax.rem(my - t + 2 * N_DEV, N_DEV)

        barrier_sem = pltpu.get_barrier_semaphore()
        for nbr in (left, right):
            pl.semaphore_signal(barrier_sem, inc=1, device_id=(nbr,),
                                device_id_type=pl.DeviceIdType.MESH)
        pl.semaphore_wait(barrier_sem, 2)

        xsend(0, x_ref)
        pown[:, :] = compute_partial(x_ref[:, :, :].reshape(R, D),
                                     chunk_of(0))

        xwait_recv(1)
        xdrain()
        xsend(1, xslots.at[1 % NXS])
        accs[:, :] = compute_partial(
            xslots[1 % NXS][:, :, :].reshape(R, D), chunk_of(1))
        acc_send(1, accs)

        def step(t, do_xfwd):
            xs = lax.rem(t, NXS) if not isinstance(t, int) else t % NXS
            xwait_recv(t)
            if do_xfwd:
                xdrain()
                xsend(t, xslots.at[xs])
            partial = compute_partial(xslots[xs][:, :, :].reshape(R, D),
                                      chunk_of(t))
            acc_wait_recv(t)
            rsbuf[t - 2, :, :] = rsbuf[t - 2] + partial
            acc_drain()
            acc_send(t, rsbuf.at[t - 2])
            return 0

        lax.fori_loop(2, 15, lambda t, _: step(t, True), 0)
        step(15, False)

        acc_wait_recv(16)
        out_ref[:, :, :] = (rsbuf[N_DEV - 2] + pown[:, :]).reshape(B_loc, Sq, D)
        acc_drain()
        xdrain()

        @functools.partial(pl.run_scoped,
                           second_barrier=pltpu.SemaphoreType.REGULAR)
        def _(second_barrier):
            for nbr in (left, right):
                pl.semaphore_signal(second_barrier, inc=1, device_id=(nbr,),
                                    device_id_type=pl.DeviceIdType.MESH)
            pl.semaphore_wait(second_barrier, 2)

    grid_spec = pltpu.PrefetchScalarGridSpec(
        num_scalar_prefetch=0,
        in_specs=[
            pl.BlockSpec(memory_space=pltpu.VMEM),
            pl.BlockSpec(memory_space=pltpu.VMEM),
            pl.BlockSpec(memory_space=pltpu.VMEM),
            pl.BlockSpec(memory_space=pltpu.VMEM),
            pl.BlockSpec(memory_space=pltpu.VMEM),
        ],
        out_specs=pl.BlockSpec(memory_space=pltpu.VMEM),
        scratch_shapes=[
            pltpu.VMEM((NXS, B_loc, Sq, D), jnp.float32),
            pltpu.VMEM((N_DEV - 1, R, D), jnp.float32),
            pltpu.VMEM((D, DP), jnp.float32),
            pltpu.VMEM((DP, D), jnp.float32),
            pltpu.VMEM((R, DP), jnp.float32),
            pltpu.VMEM((R, D), jnp.float32),
            pltpu.VMEM((R, D), jnp.float32),
            pltpu.SemaphoreType.DMA,
            pltpu.SemaphoreType.DMA,
            pltpu.SemaphoreType.DMA((N_DEV - 1,)),
            pltpu.SemaphoreType.DMA((N_DEV - 1,)),
        ],
    )

    my = lax.axis_index("i")
    Ksl = lax.dynamic_slice_in_dim(K_ext, my * H_loc, H_loc, axis=2)
    Vsl = lax.dynamic_slice_in_dim(V_ext, my * H_loc, H_loc, axis=2)
    Kt = jnp.transpose(Ksl, (0, 2, 1, 3)).reshape(B * H_loc, Skv, Dh)
    Vt = jnp.transpose(Vsl, (0, 2, 1, 3)).reshape(B * H_loc, Skv, Dh)

    return pl.pallas_call(
        body,
        out_shape=jax.ShapeDtypeStruct((B_loc, Sq, D), jnp.float32),
        grid_spec=grid_spec,
        compiler_params=pltpu.CompilerParams(
            collective_id=0, vmem_limit_bytes=64 * 1024 * 1024),
    )(x, Wq, Wo, Kt, Vt)


# device time: 520411 ns/iter; 1.0017x vs baseline; 1.0017x over previous
import functools

import jax
import jax.numpy as jnp
from jax import lax
from jax.experimental import pallas as pl
from jax.experimental.pallas import tpu as pltpu

N_DEV = 16
NXS = 5
NKS = 3


def kernel(x, Wq, Wo, K_ext, V_ext):
    B_loc, Sq, D = x.shape
    B, Skv, _, Dh = K_ext.shape
    H_loc = Wq.shape[1] // Dh
    R = B_loc * Sq
    Dp = 128
    DP = H_loc * Dp
    scale = 1.0 / (Dh ** 0.5)
    f32 = jnp.float32

    def body(x_ref, wq_ref, wo_ref, k_ref, v_ref, out_ref,
             xslots, rsbuf, wqpad, wopad, obufpad,
             accs, pown,
             x_send_sem, acc_send_sem, x_recv_sems, rs_recv_sems):
        my = lax.axis_index("i")
        left = lax.rem(my + N_DEV - 1, N_DEV)
        right = lax.rem(my + 1, N_DEV)

        wqpad[:, :] = jnp.zeros((D, DP), f32)
        wopad[:, :] = jnp.zeros((DP, D), f32)
        obufpad[:, :] = jnp.zeros((R, DP), f32)
        for h in range(H_loc):
            wqpad[:, h * Dp:h * Dp + Dh] = wq_ref[:, h * Dh:(h + 1) * Dh]
            wopad[h * Dp:h * Dp + Dh, :] = wo_ref[h * Dh:(h + 1) * Dh, :]

        def compute_partial(xc, c):
            qpad = jnp.dot(xc, wqpad[:, :], preferred_element_type=f32)
            for b in range(B_loc):
                kb = k_ref[c * B_loc + b]
                vb = v_ref[c * B_loc + b]
                for h in range(H_loc):
                    qbh = qpad[b * Sq:(b + 1) * Sq, h * Dp:h * Dp + Dh]
                    kbh = kb[:, h * Dh:(h + 1) * Dh]
                    vbh = vb[:, h * Dh:(h + 1) * Dh]
                    sc = lax.dot_general(
                        qbh, kbh, (((1,), (1,)), ((), ())),
                        preferred_element_type=f32) * scale
                    m = jnp.max(sc, axis=1, keepdims=True)
                    p = jnp.exp(sc - m)
                    l = jnp.sum(p, axis=1, keepdims=True)
                    o = jnp.dot(p, vbh, preferred_element_type=f32) / l
                    obufpad[b * Sq:(b + 1) * Sq, h * Dp:h * Dp + Dh] = o
            return jnp.dot(obufpad[:, :], wopad[:, :],
                           preferred_element_type=f32)

        def xsend(t, src):
            rdma = pltpu.make_async_remote_copy(
                src_ref=src,
                dst_ref=xslots.at[lax.rem(t + 1, NXS)],
                send_sem=x_send_sem,
                recv_sem=x_recv_sems.at[t],
                device_id=(right,), device_id_type=pl.DeviceIdType.MESH)
            rdma.start()
            return rdma

        def xwait_recv(t):
            pltpu.make_async_remote_copy(
                src_ref=xslots.at[lax.rem(t, NXS)],
                dst_ref=xslots.at[lax.rem(t, NXS)],
                send_sem=x_send_sem, recv_sem=x_recv_sems.at[t - 1],
                device_id=(left,), device_id_type=pl.DeviceIdType.MESH,
            ).wait_recv()

        def xdrain():
            pltpu.make_async_remote_copy(
                src_ref=xslots.at[0], dst_ref=xslots.at[0],
                send_sem=x_send_sem, recv_sem=x_recv_sems.at[0],
                device_id=(right,), device_id_type=pl.DeviceIdType.MESH,
            ).wait_send()

        def acc_send(t, src):
            rdma = pltpu.make_async_remote_copy(
                src_ref=src, dst_ref=rsbuf.at[t - 1],
                send_sem=acc_send_sem, recv_sem=rs_recv_sems.at[t - 1],
                device_id=(right,), device_id_type=pl.DeviceIdType.MESH)
            rdma.start()

        def acc_wait_recv(t):
            pltpu.make_async_remote_copy(
                src_ref=rsbuf.at[t - 2], dst_ref=rsbuf.at[t - 2],
                send_sem=acc_send_sem, recv_sem=rs_recv_sems.at[t - 2],
                device_id=(left,), device_id_type=pl.DeviceIdType.MESH,
            ).wait_recv()

        def acc_drain():
            pltpu.make_async_remote_copy(
                src_ref=accs, dst_ref=accs,
                send_sem=acc_send_sem, recv_sem=rs_recv_sems.at[0],
                device_id=(right,), device_id_type=pl.DeviceIdType.MESH,
            ).wait_send()

        def chunk_of(t):
            return lax.rem(my - t + 2 * N_DEV, N_DEV)

        barrier_sem = pltpu.get_barrier_semaphore()
        for nbr in (left, right):
            pl.semaphore_signal(barrier_sem, inc=1, device_id=(nbr,),
                                device_id_type=pl.DeviceIdType.MESH)
        pl.semaphore_wait(barrier_sem, 2)

        xsend(0, x_ref)
        pown[:, :] = compute_partial(x_ref[:, :, :].reshape(R, D),
                                     chunk_of(0))

        xwait_recv(1)
        xdrain()
        xsend(1, xslots.at[1 % NXS])
        accs[:, :] = compute_partial(
            xslots[1 % NXS][:, :, :].reshape(R, D), chunk_of(1))
        acc_send(1, accs)

        def step(t, do_xfwd):
            xs = lax.rem(t, NXS) if not isinstance(t, int) else t % NXS
            xwait_recv(t)
            if do_xfwd:
                xdrain()
                xsend(t, xslots.at[xs])
            partial = compute_partial(xslots[xs][:, :, :].reshape(R, D),
                                      chunk_of(t))
            acc_wait_recv(t)
            rsbuf[t - 2, :, :] = rsbuf[t - 2] + partial
            acc_drain()
            acc_send(t, rsbuf.at[t - 2])
            return 0

        lax.fori_loop(2, 15, lambda t, _: step(t, True), 0)
        step(15, False)

        acc_wait_recv(16)
        out_ref[:, :, :] = (rsbuf[N_DEV - 2] + pown[:, :]).reshape(B_loc, Sq, D)
        acc_drain()
        xdrain()

        @functools.partial(pl.run_scoped,
                           second_barrier=pltpu.SemaphoreType.REGULAR)
        def _(second_barrier):
            for nbr in (left, right):
                pl.semaphore_signal(second_barrier, inc=1, device_id=(nbr,),
                                    device_id_type=pl.DeviceIdType.MESH)
            pl.semaphore_wait(second_barrier, 2)

    grid_spec = pltpu.PrefetchScalarGridSpec(
        num_scalar_prefetch=0,
        in_specs=[
            pl.BlockSpec(memory_space=pltpu.VMEM),
            pl.BlockSpec(memory_space=pltpu.VMEM),
            pl.BlockSpec(memory_space=pltpu.VMEM),
            pl.BlockSpec(memory_space=pltpu.VMEM),
            pl.BlockSpec(memory_space=pltpu.VMEM),
        ],
        out_specs=pl.BlockSpec(memory_space=pltpu.VMEM),
        scratch_shapes=[
            pltpu.VMEM((NXS, B_loc, Sq, D), jnp.float32),
            pltpu.VMEM((N_DEV - 1, R, D), jnp.float32),
            pltpu.VMEM((D, DP), jnp.float32),
            pltpu.VMEM((DP, D), jnp.float32),
            pltpu.VMEM((R, DP), jnp.float32),
            pltpu.VMEM((R, D), jnp.float32),
            pltpu.VMEM((R, D), jnp.float32),
            pltpu.SemaphoreType.DMA,
            pltpu.SemaphoreType.DMA,
            pltpu.SemaphoreType.DMA((N_DEV - 1,)),
            pltpu.SemaphoreType.DMA((N_DEV - 1,)),
        ],
    )

    my = lax.axis_index("i")
    Kt = lax.dynamic_slice_in_dim(K_ext, my * H_loc, H_loc, axis=2)
    Kt = Kt.reshape(B, Skv, H_loc * Dh)
    Vt = lax.dynamic_slice_in_dim(V_ext, my * H_loc, H_loc, axis=2)
    Vt = Vt.reshape(B, Skv, H_loc * Dh)

    return pl.pallas_call(
        body,
        out_shape=jax.ShapeDtypeStruct((B_loc, Sq, D), jnp.float32),
        grid_spec=grid_spec,
        compiler_params=pltpu.CompilerParams(
            collective_id=0, vmem_limit_bytes=64 * 1024 * 1024),
    )(x, Wq, Wo, Kt, Vt)
